# baseline (speedup 1.0000x reference)
import functools

import jax
import jax.numpy as jnp
from jax import lax
from jax.experimental import pallas as pl
from jax.experimental.pallas import tpu as pltpu
from jax.experimental.pallas import tpu_sc as plsc

G = 128


_NC = 2
_NS = 16
_CHUNK = 80


def _sc_edge_body(hin, e, src, dst, out, sidx0, sidx1, didx0, didx1, hbuf0,
                  hbuf1, ebuf0, ebuf1, zbuf, agg, semi0, semi1, seme0, seme1,
                  semg0, semg1, *, n_nodes, n_feat, edges_per_tile):
    c = lax.axis_index("c")
    s = lax.axis_index("s")
    wid = c * _NS + s
    row0 = 632
    row_last = n_nodes - (_NS - 1) * row0
    zrows = zbuf.shape[0]
    nq = n_feat // 16
    base_r = s * row0
    nblk = jnp.where(s < _NS - 1, row0 // 8, row_last // 8)

    def _zrow(r, _):
        for q in range(nq):
            zbuf[r, pl.ds(q * 16, 16)] = jnp.zeros((16,), jnp.float32)
        return 0

    lax.fori_loop(0, zrows, _zrow, 0)

    def _zcp(k, _):
        pltpu.sync_copy(zbuf, agg.at[pl.ds(base_r + k * zrows, zrows)])
        return 0

    lax.fori_loop(0, nblk, _zcp, 0)
    plsc.subcore_barrier()

    n_chunks = edges_per_tile // _CHUNK
    slots = (
        (sidx0, didx0, hbuf0, ebuf0, semi0, seme0, semg0),
        (sidx1, didx1, hbuf1, ebuf1, semi1, seme1, semg1),
    )

    def _base(j):
        return wid * edges_per_tile + j * _CHUNK

    def _issue_loads(j, p):
        sidx, didx, hbuf, ebuf, semi, seme, semg = slots[p]
        b = _base(j)
        pltpu.async_copy(src.at[pl.ds(b, _CHUNK)], sidx, semi)
        pltpu.async_copy(dst.at[pl.ds(b, _CHUNK)], didx, semi)
        pltpu.async_copy(e.at[pl.ds(b, _CHUNK)], ebuf, seme)

    def _wait_idx(j, p):
        sidx, didx, hbuf, ebuf, semi, seme, semg = slots[p]
        b = _base(j)
        pltpu.make_async_copy(src.at[pl.ds(b, _CHUNK)], sidx, semi).wait()
        pltpu.make_async_copy(dst.at[pl.ds(b, _CHUNK)], didx, semi).wait()

    def _issue_gather(p):
        sidx, didx, hbuf, ebuf, semi, seme, semg = slots[p]
        pltpu.async_copy(hin.at[sidx], hbuf, semg)

    def _compute_scatter(j, p):
        sidx, didx, hbuf, ebuf, semi, seme, semg = slots[p]
        b = _base(j)
        pltpu.make_async_copy(e.at[pl.ds(b, _CHUNK)], ebuf, seme).wait()
        pltpu.make_async_copy(hin.at[sidx], hbuf, semg).wait()

        def _row(r, _):
            for rr in range(2):
                for q in range(nq):
                    sl = pl.ds(q * 16, 16)
                    hbuf[2 * r + rr, sl] = jnp.maximum(
                        hbuf[2 * r + rr, sl] + ebuf[2 * r + rr, sl], 0.0)
            return 0

        lax.fori_loop(0, _CHUNK // 2, _row, 0)
        pltpu.sync_copy(hbuf, agg.at[didx], add=True)

    _issue_loads(0, 0)
    _wait_idx(0, 0)
    _issue_gather(0)
    _issue_loads(1, 1)

    def _pair(i, _):
        j = 2 * i
        _wait_idx(j + 1, 1)
        _issue_gather(1)
        _compute_scatter(j, 0)
        _issue_loads(j + 2, 0)

        _wait_idx(j + 2, 0)
        _issue_gather(0)
        _compute_scatter(j + 1, 1)

        @pl.when(j + 3 < n_chunks)
        def _():
            _issue_loads(j + 3, 1)

        return 0

    lax.fori_loop(0, (n_chunks - 1) // 2, _pair, 0)
    _compute_scatter(n_chunks - 1, 0)
    plsc.subcore_barrier()

    def _ocp(k, _):
        pltpu.sync_copy(agg.at[pl.ds(base_r + k * zrows, zrows)],
                        out.at[c, pl.ds(base_r + k * zrows, zrows)])
        return 0

    lax.fori_loop(0, nblk, _ocp, 0)


@functools.partial(jax.jit, static_argnames=("n_nodes", "n_feat"))
def _sc_edge(hin, e, src, dst, *, n_nodes, n_feat):
    n_edges = src.shape[0]
    edges_per_tile = n_edges // (_NC * _NS)
    mesh = plsc.VectorSubcoreMesh(core_axis_name="c", subcore_axis_name="s")
    body = functools.partial(
        _sc_edge_body,
        n_nodes=n_nodes, n_feat=n_feat, edges_per_tile=edges_per_tile)
    f = pl.kernel(
        body,
        out_type=jax.ShapeDtypeStruct((_NC, n_nodes, n_feat), jnp.float32),
        mesh=mesh,
        scratch_types=(
            [pltpu.VMEM((_CHUNK,), jnp.int32)] * 4
            + [pltpu.VMEM((_CHUNK, n_feat), jnp.float32)] * 4
            + [pltpu.VMEM((8, n_feat), jnp.float32),
               pltpu.VMEM_SHARED((n_nodes, n_feat), jnp.float32)]
            + [pltpu.SemaphoreType.DMA] * 6
        ),
    )
    return f(hin, e, src, dst)


def _bn_in(z, g, b, n):
    m = jnp.sum(z, axis=0, keepdims=True) * (1.0 / n)
    d = z - m
    v = jnp.sum(d * d, axis=0, keepdims=True) * (1.0 / n)
    return d / jnp.sqrt(v + 1e-5) * g + b


def _edge_mlp_body(ea_ref, Wbd_ref, bbd_ref, o0_ref, o1_ref, o2_ref):
    ea = ea_ref[...]
    outs = (o0_ref, o1_ref, o2_ref)
    for l in range(3):
        outs[l][...] = (
            jnp.dot(ea, Wbd_ref[l], preferred_element_type=jnp.float32)
            + bbd_ref[pl.ds(l, 1), :])


def _edge_mlp(edge_attr, We, be):
    n_edges, de = edge_attr.shape
    h = We.shape[2]
    pk = 128 // de
    Wbd = jax.vmap(lambda w: jnp.kron(jnp.eye(pk, dtype=w.dtype), w))(We)
    bbd = jnp.tile(be, (1, pk))
    ea8 = edge_attr.reshape(n_edges // pk, pk * de)
    blk = 400
    grid = ea8.shape[0] // blk
    out_sd = jax.ShapeDtypeStruct((n_edges // pk, pk * h), jnp.float32)
    outs = pl.pallas_call(
        _edge_mlp_body,
        grid=(grid,),
        in_specs=[
            pl.BlockSpec((blk, pk * de), lambda i: (i, 0)),
            pl.BlockSpec((3, pk * de, pk * h), lambda i: (0, 0, 0)),
            pl.BlockSpec((3, pk * h), lambda i: (0, 0)),
        ],
        out_specs=[
            pl.BlockSpec((blk, pk * h), lambda i: (i, 0)),
            pl.BlockSpec((blk, pk * h), lambda i: (i, 0)),
            pl.BlockSpec((blk, pk * h), lambda i: (i, 0)),
        ],
        out_shape=[out_sd, out_sd, out_sd],
    )(ea8, Wbd, bbd)
    return [o.reshape(n_edges, h) for o in outs]


def _pre_body(x_ref, w_ref, b_ref, o_ref):
    o_ref[...] = (
        jnp.dot(x_ref[...], w_ref[...], preferred_element_type=jnp.float32)
        + b_ref[...])


def _tc_pre(x, W_in, b_in):
    n, _ = x.shape
    h = W_in.shape[1]
    return pl.pallas_call(
        _pre_body,
        out_shape=jax.ShapeDtypeStruct((n, h), jnp.float32),
    )(x, W_in, b_in.reshape(1, h))


def _mid_body(hin_ref, agg_ref, batch_ref, vn_ref, w1_ref, b1_ref, g1_ref,
              be1_ref, w2_ref, b2_ref, gbn_ref, bbn_ref, vw1_ref, vb1_ref,
              vg1_ref, vbe1_ref, vw2_ref, vb2_ref, vg2_ref, vbe2_ref,
              eps_ref, hin_next_ref, vn_next_ref, *, n, g_graphs):
    hin = hin_ref[...]
    z = (1.0 + eps_ref[0, 0]) * hin + agg_ref[0] + agg_ref[1]
    z1 = jnp.dot(z, w1_ref[...], preferred_element_type=jnp.float32) + b1_ref[...]
    z1 = _bn_in(z1, g1_ref[...], be1_ref[...], n)
    z1 = jnp.maximum(z1, 0.0)
    z2 = jnp.dot(z1, w2_ref[...], preferred_element_type=jnp.float32) + b2_ref[...]
    z2 = _bn_in(z2, gbn_ref[...], bbn_ref[...], n)
    z2 = jnp.maximum(z2, 0.0)
    h_new = z2 + hin
    onehot = (batch_ref[...] ==
              lax.broadcasted_iota(jnp.int32, (n, g_graphs), 1)
              ).astype(jnp.float32)
    segsum = lax.dot_general(onehot, hin, (((0,), (0,)), ((), ())),
                             preferred_element_type=jnp.float32)
    vtmp = segsum + vn_ref[...]
    v = jnp.dot(vtmp, vw1_ref[...], preferred_element_type=jnp.float32) + vb1_ref[...]
    v = _bn_in(v, vg1_ref[...], vbe1_ref[...], g_graphs)
    v = jnp.maximum(v, 0.0)
    v = jnp.dot(v, vw2_ref[...], preferred_element_type=jnp.float32) + vb2_ref[...]
    v = _bn_in(v, vg2_ref[...], vbe2_ref[...], g_graphs)
    vn_next = jnp.maximum(v, 0.0)
    vn_next_ref[...] = vn_next
    hin_next_ref[...] = h_new + jnp.dot(onehot, vn_next,
                                        preferred_element_type=jnp.float32)


def _tc_mid(hin, agg2, batch2, vn, Wl, eps_l):
    n, h = hin.shape
    (w1, b1, g1, be1, w2, b2, gbn, bbn,
     vw1, vb1, vg1, vbe1, vw2, vb2, vg2, vbe2) = Wl
    n_in = 21
    specs = ([pl.BlockSpec(memory_space=pltpu.VMEM)] * (n_in - 1)
             + [pl.BlockSpec(memory_space=pltpu.SMEM)])
    return pl.pallas_call(
        functools.partial(_mid_body, n=n, g_graphs=G),
        in_specs=specs,
        out_shape=[
            jax.ShapeDtypeStruct((n, h), jnp.float32),
            jax.ShapeDtypeStruct((G, h), jnp.float32),
        ],
    )(hin, agg2, batch2, vn, w1, b1, g1, be1, w2, b2, gbn, bbn,
      vw1, vb1, vg1, vbe1, vw2, vb2, vg2, vbe2, eps_l)


def _last_body(hin_ref, agg_ref, batch_ref, w1_ref, b1_ref, g1_ref, be1_ref,
               w2_ref, b2_ref, gbn_ref, bbn_ref, eps_ref,
               hnode_ref, hgraph_ref, *, n, g_graphs):
    hin = hin_ref[...]
    z = (1.0 + eps_ref[0, 0]) * hin + agg_ref[0] + agg_ref[1]
    z1 = jnp.dot(z, w1_ref[...], preferred_element_type=jnp.float32) + b1_ref[...]
    z1 = _bn_in(z1, g1_ref[...], be1_ref[...], n)
    z1 = jnp.maximum(z1, 0.0)
    z2 = jnp.dot(z1, w2_ref[...], preferred_element_type=jnp.float32) + b2_ref[...]
    z2 = _bn_in(z2, gbn_ref[...], bbn_ref[...], n)
    h_node = z2 + hin
    hnode_ref[...] = h_node
    batch = batch_ref[...]
    neg_inf = jnp.float32(-jnp.inf)

    def _seg(gi, _):
        mask = batch == gi
        vals = jnp.where(mask, h_node, neg_inf)
        hgraph_ref[pl.ds(gi, 1), :] = jnp.max(vals, axis=0, keepdims=True)
        return 0

    lax.fori_loop(0, g_graphs, _seg, 0)


def _tc_last(hin, agg2, batch2, Wl, eps_l):
    n, h = hin.shape
    w1, b1, g1, be1, w2, b2, gbn, bbn = Wl
    n_in = 12
    specs = ([pl.BlockSpec(memory_space=pltpu.VMEM)] * (n_in - 1)
             + [pl.BlockSpec(memory_space=pltpu.SMEM)])
    return pl.pallas_call(
        functools.partial(_last_body, n=n, g_graphs=G),
        in_specs=specs,
        out_shape=[
            jax.ShapeDtypeStruct((n, h), jnp.float32),
            jax.ShapeDtypeStruct((G, h), jnp.float32),
        ],
    )(hin, agg2, batch2, w1, b1, g1, be1, w2, b2, gbn, bbn, eps_l)


def kernel(x, edge_index, edge_attr, batch, W_in, b_in, eps, We, be, W1, b1,
           g1, be1, W2, b2, gbn, bbn, Vw1, Vb1, Vg1, Vbe1, Vw2, Vb2, Vg2,
           Vbe2):
    n, _ = x.shape
    h = W_in.shape[1]
    src = edge_index[0]
    dst = edge_index[1]
    batch2 = batch.reshape(n, 1)

    e_all = _edge_mlp(edge_attr, We, be)
    hin = _tc_pre(x, W_in, b_in)
    vn = jnp.zeros((G, h), jnp.float32)

    def row(a):
        return a.reshape(1, -1)

    for l in range(3):
        agg2 = _sc_edge(hin, e_all[l], src, dst, n_nodes=n, n_feat=h)
        eps_l = eps[l].reshape(1, 1)
        if l < 2:
            Wl = (W1[l], row(b1[l]), row(g1[l]), row(be1[l]), W2[l],
                  row(b2[l]), row(gbn[l]), row(bbn[l]),
                  Vw1[l], row(Vb1[l]), row(Vg1[l]), row(Vbe1[l]), Vw2[l],
                  row(Vb2[l]), row(Vg2[l]), row(Vbe2[l]))
            hin, vn = _tc_mid(hin, agg2, batch2, vn, Wl, eps_l)
        else:
            Wl = (W1[l], row(b1[l]), row(g1[l]), row(be1[l]), W2[l],
                  row(b2[l]), row(gbn[l]), row(bbn[l]))
            h_node, h_graph = _tc_last(hin, agg2, batch2, Wl, eps_l)
    return (h_graph, h_node)


# speedup vs baseline: 1.2416x; 1.2416x over previous
import functools

import jax
import jax.numpy as jnp
from jax import lax
from jax.experimental import pallas as pl
from jax.experimental.pallas import tpu as pltpu
from jax.experimental.pallas import tpu_sc as plsc

G = 128


_NC = 2
_NS = 16
_CHUNK = 80


def _sc_edge_body(hin, e, src, dst, out, sidx0, sidx1, didx0, didx1, hbuf0,
                  hbuf1, ebuf0, ebuf1, zbuf, agg, semi0, semi1, seme0, seme1,
                  semg0, semg1, sems0, sems1, *, n_nodes, n_feat,
                  edges_per_tile):
    c = lax.axis_index("c")
    s = lax.axis_index("s")
    wid = c * _NS + s
    row0 = 632
    row_last = n_nodes - (_NS - 1) * row0
    zrows = zbuf.shape[0]
    nq = n_feat // 16
    base_r = s * row0
    nblk = jnp.where(s < _NS - 1, row0 // 8, row_last // 8)

    def _zrow(r, _):
        for q in range(nq):
            zbuf[r, pl.ds(q * 16, 16)] = jnp.zeros((16,), jnp.float32)
        return 0

    lax.fori_loop(0, zrows, _zrow, 0)

    def _zcp(k, _):
        pltpu.sync_copy(zbuf, agg.at[pl.ds(base_r + k * zrows, zrows)])
        return 0

    lax.fori_loop(0, nblk, _zcp, 0)
    plsc.subcore_barrier()

    n_chunks = edges_per_tile // _CHUNK
    half = _CHUNK // 2
    slots = (
        (sidx0, didx0, hbuf0, ebuf0, semi0, seme0, semg0, sems0),
        (sidx1, didx1, hbuf1, ebuf1, semi1, seme1, semg1, sems1),
    )

    def _base(j):
        return wid * edges_per_tile + j * _CHUNK

    def _issue_loads(j, p):
        sidx, didx, hbuf, ebuf, semi, seme, semg, sems = slots[p]
        b = _base(j)
        pltpu.async_copy(src.at[pl.ds(b, _CHUNK)], sidx, semi)
        pltpu.async_copy(dst.at[wid, j], didx, semi)
        pltpu.async_copy(e.at[pl.ds(b, _CHUNK)], ebuf, seme)

    def _wait_idx(j, p):
        sidx, didx, hbuf, ebuf, semi, seme, semg, sems = slots[p]
        b = _base(j)
        pltpu.make_async_copy(src.at[pl.ds(b, _CHUNK)], sidx, semi).wait()
        pltpu.make_async_copy(dst.at[wid, j], didx, semi).wait()

    def _issue_gather(p):
        sidx, didx, hbuf, ebuf, semi, seme, semg, sems = slots[p]
        pltpu.async_copy(hin.at[sidx], hbuf, semg)

    def _wait_scatter(p):
        sidx, didx, hbuf, ebuf, semi, seme, semg, sems = slots[p]
        for hh in range(2):
            pltpu.make_async_copy(hbuf.at[pl.ds(hh * half, half)],
                                  agg.at[didx.at[hh]], sems).wait()

    def _compute(j, p):
        sidx, didx, hbuf, ebuf, semi, seme, semg, sems = slots[p]
        b = _base(j)
        pltpu.make_async_copy(e.at[pl.ds(b, _CHUNK)], ebuf, seme).wait()
        pltpu.make_async_copy(hin.at[sidx], hbuf, semg).wait()

        def _row(r, _):
            for rr in range(2):
                for q in range(nq):
                    sl = pl.ds(q * 16, 16)
                    hbuf[2 * r + rr, sl] = jnp.maximum(
                        hbuf[2 * r + rr, sl] + ebuf[2 * r + rr, sl], 0.0)
            return 0

        for hh in range(2):
            lax.fori_loop(hh * (half // 2), (hh + 1) * (half // 2), _row, 0)
            pltpu.async_copy(hbuf.at[pl.ds(hh * half, half)],
                             agg.at[didx.at[hh]], sems, add=True)

    _issue_loads(0, 0)
    _wait_idx(0, 0)
    _issue_gather(0)
    _issue_loads(1, 1)

    def _pair(i, _):
        j = 2 * i
        _wait_idx(j + 1, 1)

        @pl.when(i > 0)
        def _():
            _wait_scatter(1)

        _issue_gather(1)
        _compute(j, 0)
        _issue_loads(j + 2, 0)

        _wait_idx(j + 2, 0)
        _wait_scatter(0)
        _issue_gather(0)
        _compute(j + 1, 1)

        @pl.when(j + 3 < n_chunks)
        def _():
            _issue_loads(j + 3, 1)

        return 0

    lax.fori_loop(0, (n_chunks - 1) // 2, _pair, 0)
    _compute(n_chunks - 1, 0)
    _wait_scatter(1)
    _wait_scatter(0)
    plsc.subcore_barrier()

    def _ocp(k, _):
        pltpu.sync_copy(agg.at[pl.ds(base_r + k * zrows, zrows)],
                        out.at[c, pl.ds(base_r + k * zrows, zrows)])
        return 0

    lax.fori_loop(0, nblk, _ocp, 0)


@functools.partial(jax.jit, static_argnames=("n_nodes", "n_feat"))
def _sc_edge(hin, e, src, dst, *, n_nodes, n_feat):
    n_edges = src.shape[0]
    edges_per_tile = n_edges // (_NC * _NS)
    n_chunks = edges_per_tile // _CHUNK
    dst4 = dst.reshape(_NC * _NS, n_chunks, 2, _CHUNK // 2)
    mesh = plsc.VectorSubcoreMesh(core_axis_name="c", subcore_axis_name="s")
    body = functools.partial(
        _sc_edge_body,
        n_nodes=n_nodes, n_feat=n_feat, edges_per_tile=edges_per_tile)
    f = pl.kernel(
        body,
        out_type=jax.ShapeDtypeStruct((_NC, n_nodes, n_feat), jnp.float32),
        mesh=mesh,
        scratch_types=(
            [pltpu.VMEM((_CHUNK,), jnp.int32)] * 2
            + [pltpu.VMEM((2, _CHUNK // 2), jnp.int32)] * 2
            + [pltpu.VMEM((_CHUNK, n_feat), jnp.float32)] * 4
            + [pltpu.VMEM((8, n_feat), jnp.float32),
               pltpu.VMEM_SHARED((n_nodes, n_feat), jnp.float32)]
            + [pltpu.SemaphoreType.DMA] * 8
        ),
    )
    return f(hin, e, src, dst4)


def _bn_in(z, g, b, n):
    m = jnp.sum(z, axis=0, keepdims=True) * (1.0 / n)
    d = z - m
    v = jnp.sum(d * d, axis=0, keepdims=True) * (1.0 / n)
    return d / jnp.sqrt(v + 1e-5) * g + b


def _edge_mlp_body(ea_ref, We_ref, be_ref, o0_ref, o1_ref, o2_ref):
    ea = ea_ref[...]
    outs = (o0_ref, o1_ref, o2_ref)
    for l in range(3):
        outs[l][...] = (
            jnp.dot(ea, We_ref[l], preferred_element_type=jnp.float32)
            + be_ref[pl.ds(l, 1), :])


def _edge_mlp(edge_attr, We, be):
    n_edges, de = edge_attr.shape
    h = We.shape[2]
    blk = 3200
    grid = n_edges // blk
    out_sd = jax.ShapeDtypeStruct((n_edges, h), jnp.float32)
    return pl.pallas_call(
        _edge_mlp_body,
        grid=(grid,),
        in_specs=[
            pl.BlockSpec((blk, de), lambda i: (i, 0)),
            pl.BlockSpec((3, de, h), lambda i: (0, 0, 0)),
            pl.BlockSpec((3, h), lambda i: (0, 0)),
        ],
        out_specs=[
            pl.BlockSpec((blk, h), lambda i: (i, 0)),
            pl.BlockSpec((blk, h), lambda i: (i, 0)),
            pl.BlockSpec((blk, h), lambda i: (i, 0)),
        ],
        out_shape=[out_sd, out_sd, out_sd],
    )(edge_attr, We, be)


def _pre_body(x_ref, w_ref, b_ref, o_ref):
    o_ref[...] = (
        jnp.dot(x_ref[...], w_ref[...], preferred_element_type=jnp.float32)
        + b_ref[...])


def _tc_pre(x, W_in, b_in):
    n, _ = x.shape
    h = W_in.shape[1]
    return pl.pallas_call(
        _pre_body,
        out_shape=jax.ShapeDtypeStruct((n, h), jnp.float32),
    )(x, W_in, b_in.reshape(1, h))


def _mid_body(hin_ref, agg_ref, batch_ref, vn_ref, w1_ref, b1_ref, g1_ref,
              be1_ref, w2_ref, b2_ref, gbn_ref, bbn_ref, vw1_ref, vb1_ref,
              vg1_ref, vbe1_ref, vw2_ref, vb2_ref, vg2_ref, vbe2_ref,
              eps_ref, hin_next_ref, vn_next_ref, *, n, g_graphs):
    hin = hin_ref[...]
    z = (1.0 + eps_ref[0, 0]) * hin + agg_ref[0] + agg_ref[1]
    z1 = jnp.dot(z, w1_ref[...], preferred_element_type=jnp.float32) + b1_ref[...]
    z1 = _bn_in(z1, g1_ref[...], be1_ref[...], n)
    z1 = jnp.maximum(z1, 0.0)
    z2 = jnp.dot(z1, w2_ref[...], preferred_element_type=jnp.float32) + b2_ref[...]
    z2 = _bn_in(z2, gbn_ref[...], bbn_ref[...], n)
    z2 = jnp.maximum(z2, 0.0)
    h_new = z2 + hin
    onehot = (batch_ref[...] ==
              lax.broadcasted_iota(jnp.int32, (n, g_graphs), 1)
              ).astype(jnp.float32)
    segsum = lax.dot_general(onehot, hin, (((0,), (0,)), ((), ())),
                             preferred_element_type=jnp.float32)
    vtmp = segsum + vn_ref[...]
    v = jnp.dot(vtmp, vw1_ref[...], preferred_element_type=jnp.float32) + vb1_ref[...]
    v = _bn_in(v, vg1_ref[...], vbe1_ref[...], g_graphs)
    v = jnp.maximum(v, 0.0)
    v = jnp.dot(v, vw2_ref[...], preferred_element_type=jnp.float32) + vb2_ref[...]
    v = _bn_in(v, vg2_ref[...], vbe2_ref[...], g_graphs)
    vn_next = jnp.maximum(v, 0.0)
    vn_next_ref[...] = vn_next
    hin_next_ref[...] = h_new + jnp.dot(onehot, vn_next,
                                        preferred_element_type=jnp.float32)


def _tc_mid(hin, agg2, batch2, vn, Wl, eps_l):
    n, h = hin.shape
    (w1, b1, g1, be1, w2, b2, gbn, bbn,
     vw1, vb1, vg1, vbe1, vw2, vb2, vg2, vbe2) = Wl
    n_in = 21
    specs = ([pl.BlockSpec(memory_space=pltpu.VMEM)] * (n_in - 1)
             + [pl.BlockSpec(memory_space=pltpu.SMEM)])
    return pl.pallas_call(
        functools.partial(_mid_body, n=n, g_graphs=G),
        in_specs=specs,
        out_shape=[
            jax.ShapeDtypeStruct((n, h), jnp.float32),
            jax.ShapeDtypeStruct((G, h), jnp.float32),
        ],
    )(hin, agg2, batch2, vn, w1, b1, g1, be1, w2, b2, gbn, bbn,
      vw1, vb1, vg1, vbe1, vw2, vb2, vg2, vbe2, eps_l)


def _last_body(hin_ref, agg_ref, batch_ref, w1_ref, b1_ref, g1_ref, be1_ref,
               w2_ref, b2_ref, gbn_ref, bbn_ref, eps_ref,
               hnode_ref, hgraph_ref, *, n, g_graphs):
    hin = hin_ref[...]
    z = (1.0 + eps_ref[0, 0]) * hin + agg_ref[0] + agg_ref[1]
    z1 = jnp.dot(z, w1_ref[...], preferred_element_type=jnp.float32) + b1_ref[...]
    z1 = _bn_in(z1, g1_ref[...], be1_ref[...], n)
    z1 = jnp.maximum(z1, 0.0)
    z2 = jnp.dot(z1, w2_ref[...], preferred_element_type=jnp.float32) + b2_ref[...]
    z2 = _bn_in(z2, gbn_ref[...], bbn_ref[...], n)
    h_node = z2 + hin
    hnode_ref[...] = h_node
    batch = batch_ref[...]
    neg_inf = jnp.float32(-jnp.inf)

    def _seg(gi, _):
        mask = batch == gi
        vals = jnp.where(mask, h_node, neg_inf)
        hgraph_ref[pl.ds(gi, 1), :] = jnp.max(vals, axis=0, keepdims=True)
        return 0

    lax.fori_loop(0, g_graphs, _seg, 0)


def _tc_last(hin, agg2, batch2, Wl, eps_l):
    n, h = hin.shape
    w1, b1, g1, be1, w2, b2, gbn, bbn = Wl
    n_in = 12
    specs = ([pl.BlockSpec(memory_space=pltpu.VMEM)] * (n_in - 1)
             + [pl.BlockSpec(memory_space=pltpu.SMEM)])
    return pl.pallas_call(
        functools.partial(_last_body, n=n, g_graphs=G),
        in_specs=specs,
        out_shape=[
            jax.ShapeDtypeStruct((n, h), jnp.float32),
            jax.ShapeDtypeStruct((G, h), jnp.float32),
        ],
    )(hin, agg2, batch2, w1, b1, g1, be1, w2, b2, gbn, bbn, eps_l)


def kernel(x, edge_index, edge_attr, batch, W_in, b_in, eps, We, be, W1, b1,
           g1, be1, W2, b2, gbn, bbn, Vw1, Vb1, Vg1, Vbe1, Vw2, Vb2, Vg2,
           Vbe2):
    n, _ = x.shape
    h = W_in.shape[1]
    src = edge_index[0]
    dst = edge_index[1]
    batch2 = batch.reshape(n, 1)

    e_all = _edge_mlp(edge_attr, We, be)
    hin = _tc_pre(x, W_in, b_in)
    vn = jnp.zeros((G, h), jnp.float32)

    def row(a):
        return a.reshape(1, -1)

    for l in range(3):
        agg2 = _sc_edge(hin, e_all[l], src, dst, n_nodes=n, n_feat=h)
        eps_l = eps[l].reshape(1, 1)
        if l < 2:
            Wl = (W1[l], row(b1[l]), row(g1[l]), row(be1[l]), W2[l],
                  row(b2[l]), row(gbn[l]), row(bbn[l]),
                  Vw1[l], row(Vb1[l]), row(Vg1[l]), row(Vbe1[l]), Vw2[l],
                  row(Vb2[l]), row(Vg2[l]), row(Vbe2[l]))
            hin, vn = _tc_mid(hin, agg2, batch2, vn, Wl, eps_l)
        else:
            Wl = (W1[l], row(b1[l]), row(g1[l]), row(be1[l]), W2[l],
                  row(b2[l]), row(gbn[l]), row(bbn[l]))
            h_node, h_graph = _tc_last(hin, agg2, batch2, Wl, eps_l)
    return (h_graph, h_node)
